# baseline (speedup 1.0000x reference)
import functools

import jax
import jax.numpy as jnp
from jax import lax
from jax.experimental import pallas as pl
from jax.experimental.pallas import tpu as pltpu
from jax.experimental.pallas import tpu_sc as plsc

LANES = 16
OUT_CHUNK = 4096
REGIONS = [(0, 33280), (33280, 33280), (66560, 33408)]
ALIGNED_END = 99968
BUF_LEN = 33408


def _build(V, D, B):
  info = plsc.get_sparse_core_info()
  NC = info.num_cores
  NW = NC * info.num_subcores
  d_per_w = D // NW
  n_chunks = B // OUT_CHUNK
  n_regions = len(REGIONS)
  tail_len = V - ALIGNED_END

  mesh = plsc.VectorSubcoreMesh(core_axis_name="c", subcore_axis_name="s")

  @functools.partial(
      pl.kernel,
      mesh=mesh,
      out_type=jax.ShapeDtypeStruct((D, B), jnp.float32),
      compiler_params=pltpu.CompilerParams(
          use_tc_tiling_on_sc=True, needs_layout_passes=False),
      scratch_types=[
          pltpu.VMEM((BUF_LEN,), jnp.float32),
          pltpu.VMEM((BUF_LEN,), jnp.float32),
          pltpu.VMEM((B,), jnp.int32),
          pltpu.VMEM((B,), jnp.float32),
          pltpu.VMEM((B,), jnp.float32),
          pltpu.VMEM((8, tail_len), jnp.float32),
          pltpu.SemaphoreType.DMA((2,)),
          pltpu.SemaphoreType.DMA((2,)),
          pltpu.SemaphoreType.DMA,
          pltpu.SemaphoreType.DMA,
      ],
  )
  def k(tw_hbm, t_hbm, out_hbm, bufa, bufb, idx_v, o0_v, o1_v, tail_v,
        rsem, wsem, isem, tsem):
    wid = lax.axis_index("s") * NC + lax.axis_index("c")
    d0 = wid * d_per_w
    bufs = (bufa, bufb)
    o_bufs = (o0_v, o1_v)
    iota = lax.iota(jnp.int32, LANES)
    dmod0 = d0 % 8

    idx_copy = pltpu.async_copy(t_hbm, idx_v, isem)
    tail_copy = pltpu.async_copy(
        tw_hbm.at[pl.ds((d0 // 8) * 8, 8), pl.ds(ALIGNED_END, tail_len)],
        tail_v, tsem)

    loads = [(rr, p) for rr in range(d_per_w) for p in range(n_regions)]

    def start_load(g):
      rr, p = loads[g]
      base, ln = REGIONS[p]
      return pltpu.async_copy(
          tw_hbm.at[d0 + rr, pl.ds(base, ln)],
          bufs[g % 2].at[pl.ds(0, ln)],
          rsem.at[g % 2])

    inflight = {0: start_load(0)}
    idx_copy.wait()
    tail_copy.wait()

    writes = []
    for g in range(len(loads)):
      rr, p = loads[g]
      inflight.pop(g).wait()
      if g + 1 < len(loads):
        inflight[g + 1] = start_load(g + 1)
      buf = bufs[g % 2]
      o_ref = o_bufs[rr]
      base, ln = REGIONS[p]
      last_region = p == n_regions - 1

      @plsc.parallel_loop(0, B, step=LANES, unroll=8)
      def _(j, buf=buf, o_ref=o_ref, base=base, ln=ln,
            last_region=last_region):
        tvec = idx_v[pl.ds(j, LANES)]
        rel = jnp.minimum(jnp.maximum(tvec - base, 0), ln - 1)
        g_main = plsc.load_gather(buf, [rel])
        if last_region:
          mask = tvec >= base
          is_tail = tvec >= ALIGNED_END
          relt = jnp.maximum(tvec - ALIGNED_END, 0)
          g_tail = plsc.load_gather(
              tail_v, [jnp.full((LANES,), dmod0 + rr, jnp.int32), relt])
          vals = jnp.where(is_tail, g_tail, g_main)
        else:
          mask = (tvec >= base) & (tvec < base + ln)
          vals = g_main
        plsc.store_scatter(o_ref, [iota + j], vals, mask=mask)

      if last_region:
        for ci in range(n_chunks):
          writes.append(
              pltpu.async_copy(
                  o_ref.at[pl.ds(ci * OUT_CHUNK, OUT_CHUNK)],
                  out_hbm.at[d0 + rr, pl.ds(ci * OUT_CHUNK, OUT_CHUNK)],
                  wsem.at[rr]))
    for c in writes:
      c.wait()

  return k


def kernel(t, embed_weight):
  V, D = embed_weight.shape
  B = t.shape[0]
  out_t = _build(V, D, B)(embed_weight.T, t.astype(jnp.int32))
  return out_t.T


# speedup vs baseline: 1.1869x; 1.1869x over previous
import functools

import jax
import jax.numpy as jnp
from jax import lax
from jax.experimental import pallas as pl
from jax.experimental.pallas import tpu as pltpu
from jax.experimental.pallas import tpu_sc as plsc

LANES = 16
OUT_CHUNK = 4096


def _build(V, D, B):
  info = plsc.get_sparse_core_info()
  NC = info.num_cores
  NW = NC * info.num_subcores
  d_per_w = D // NW
  n_chunks = B // OUT_CHUNK

  mesh = plsc.VectorSubcoreMesh(core_axis_name="c", subcore_axis_name="s")

  @functools.partial(
      pl.kernel,
      mesh=mesh,
      out_type=jax.ShapeDtypeStruct((D, B), jnp.float32),
      compiler_params=pltpu.CompilerParams(
          use_tc_tiling_on_sc=True, needs_layout_passes=False),
      scratch_types=[
          pltpu.VMEM((V,), jnp.float32),
          pltpu.VMEM((B,), jnp.int32),
          pltpu.VMEM((OUT_CHUNK,), jnp.float32),
          pltpu.VMEM((OUT_CHUNK,), jnp.float32),
          pltpu.SemaphoreType.DMA((2,)),
          pltpu.SemaphoreType.DMA((4,)),
      ],
  )
  def k(tw_hbm, t_hbm, out_hbm, row_v, idx_v, o0_v, o1_v, wsem, qsem):
    wid = lax.axis_index("s") * NC + lax.axis_index("c")

    def load_row(d):
      pltpu.sync_copy(tw_hbm.at[d], row_v)

    idx_copy = pltpu.async_copy(t_hbm, idx_v, qsem.at[0])
    load_row(wid * d_per_w)
    idx_copy.wait()

    o_bufs = (o0_v, o1_v)
    pending = [None, None]
    for rr in range(d_per_w):
      d = wid * d_per_w + rr
      for ci in range(n_chunks):
        slot = (rr * n_chunks + ci) % 2
        if pending[slot] is not None:
          pending[slot].wait()
        o_ref = o_bufs[slot]

        @plsc.parallel_loop(0, OUT_CHUNK, step=LANES, unroll=8)
        def _(j, ci=ci, o_ref=o_ref):
          tvec = idx_v[pl.ds(ci * OUT_CHUNK + j, LANES)]
          o_ref[pl.ds(j, LANES)] = plsc.load_gather(row_v, [tvec])

        pending[slot] = pltpu.async_copy(
            o_ref, out_hbm.at[d, pl.ds(ci * OUT_CHUNK, OUT_CHUNK)],
            wsem.at[slot])
      if rr + 1 < d_per_w:
        load_row(d + 1)
    for c in pending:
      if c is not None:
        c.wait()

  return k


def kernel(t, embed_weight):
  V, D = embed_weight.shape
  B = t.shape[0]
  out_t = _build(V, D, B)(embed_weight.T, t.astype(jnp.int32))
  return out_t.T


# speedup vs baseline: 1.2960x; 1.0919x over previous
import functools

import jax
import jax.numpy as jnp
from jax import lax
from jax.experimental import pallas as pl
from jax.experimental.pallas import tpu as pltpu
from jax.experimental.pallas import tpu_sc as plsc

LANES = 16
OUT_CHUNK = 4096


def _build(V, D, B):
  info = plsc.get_sparse_core_info()
  NC = info.num_cores
  NW = NC * info.num_subcores
  d_per_w = D // NW
  n_chunks = B // OUT_CHUNK

  mesh = plsc.VectorSubcoreMesh(core_axis_name="c", subcore_axis_name="s")

  @functools.partial(
      pl.kernel,
      mesh=mesh,
      out_type=jax.ShapeDtypeStruct((D, B), jnp.float32),
      compiler_params=pltpu.CompilerParams(
          use_tc_tiling_on_sc=True, needs_layout_passes=False),
      scratch_types=[
          pltpu.VMEM((V,), jnp.float32),
          pltpu.VMEM((B,), jnp.int32),
          pltpu.VMEM((OUT_CHUNK,), jnp.float32),
          pltpu.VMEM((OUT_CHUNK,), jnp.float32),
          pltpu.VMEM_SHARED((B,), jnp.int32),
          pltpu.SemaphoreType.DMA((2,)),
          pltpu.SemaphoreType.DMA((4,)),
      ],
  )
  def k(tw_hbm, t_hbm, out_hbm, row_v, idx_v, o0_v, o1_v, sh_idx, wsem,
        qsem):
    sid = lax.axis_index("s")
    wid = sid * NC + lax.axis_index("c")

    def load_row(d):
      pltpu.sync_copy(tw_hbm.at[d], row_v)

    row_copy = pltpu.async_copy(
        tw_hbm.at[wid * d_per_w, pl.ds(0, V)], row_v, qsem.at[0])

    @pl.when(sid == 0)
    def _():
      pltpu.sync_copy(t_hbm, sh_idx)

    plsc.subcore_barrier()
    pltpu.sync_copy(sh_idx, idx_v)
    row_copy.wait()

    o_bufs = (o0_v, o1_v)
    pending = [None, None]
    for rr in range(d_per_w):
      d = wid * d_per_w + rr
      for ci in range(n_chunks):
        slot = (rr * n_chunks + ci) % 2
        if pending[slot] is not None:
          pending[slot].wait()
        o_ref = o_bufs[slot]

        @plsc.parallel_loop(0, OUT_CHUNK, step=LANES, unroll=8)
        def _(j, ci=ci, o_ref=o_ref):
          tvec = idx_v[pl.ds(ci * OUT_CHUNK + j, LANES)]
          o_ref[pl.ds(j, LANES)] = plsc.load_gather(row_v, [tvec])

        pending[slot] = pltpu.async_copy(
            o_ref, out_hbm.at[d, pl.ds(ci * OUT_CHUNK, OUT_CHUNK)],
            wsem.at[slot])
      if rr + 1 < d_per_w:
        load_row(d + 1)
    for c in pending:
      if c is not None:
        c.wait()

  return k


def kernel(t, embed_weight):
  V, D = embed_weight.shape
  B = t.shape[0]
  out_t = _build(V, D, B)(embed_weight.T, t.astype(jnp.int32))
  return out_t.T
